# baseline (speedup 1.0000x reference)
import functools

import jax
import jax.numpy as jnp
from jax import lax
from jax.experimental import pallas as pl
from jax.experimental.pallas import tpu as pltpu
from jax.experimental.pallas import tpu_sc as plsc

_TARGET_RATE = 0.5
_INT_MIN = -2147483648


def _build_mask_sc(imp_flat, B, N, k):
    info = plsc.get_sparse_core_info()
    NC, NS = info.num_cores, info.num_subcores
    WPB = (NC * NS) // B
    CH = N // WPB
    NV = CH // 16
    U = 8 if NV % 8 == 0 else 1
    NG = NV // U
    ROW = 16
    RPW = 3
    mesh = plsc.VectorSubcoreMesh(core_axis_name="c", subcore_axis_name="s")

    @functools.partial(
        pl.kernel,
        mesh=mesh,
        compiler_params=pltpu.CompilerParams(needs_layout_passes=False),
        out_type=jax.ShapeDtypeStruct((B * N,), jnp.float32),
        scratch_types=[
            pltpu.VMEM((CH,), jnp.float32),
            pltpu.VMEM((CH,), jnp.int32),
            pltpu.VMEM((CH,), jnp.float32),
            pltpu.VMEM((RPW * ROW,), jnp.int32),
            pltpu.VMEM((WPB * RPW * ROW,), jnp.int32),
            pltpu.VMEM_SHARED((2 * NS * RPW * ROW,), jnp.int32),
        ],
    )
    def sc_mask(imp_hbm, out_hbm, x_v, key_v, out_v, stage_v, team_v, counts_sm):
        c = lax.axis_index("c")
        s = lax.axis_index("s")
        batch = c * (B // NC) + s // WPB
        slot = s % WPB
        team_lo = (s // WPB) * WPB
        base = batch * N + slot * CH

        pltpu.sync_copy(imp_hbm.at[pl.ds(base, CH)], x_v)

        def keys_body(g, carry):
            for u in range(U):
                i = g * U + u
                bits = lax.bitcast_convert_type(x_v[pl.ds(i * 16, 16)],
                                                jnp.int32)
                key_v[pl.ds(i * 16, 16)] = jnp.where(
                    bits >= 0, bits, jnp.int32(_INT_MIN) - bits)
            return carry

        lax.fori_loop(0, NG, keys_body, jnp.int32(0))

        one = jnp.int32(1)
        zero16 = jnp.zeros((16,), jnp.int32)

        def publish(parity, vecs):
            for j, vec in enumerate(vecs):
                stage_v[pl.ds(j * ROW, ROW)] = vec
            off = (parity * NS + s) * (RPW * ROW)
            pltpu.sync_copy(stage_v, counts_sm.at[pl.ds(off, RPW * ROW)])
            plsc.subcore_barrier()
            toff = (parity * NS + team_lo) * (RPW * ROW)
            pltpu.sync_copy(counts_sm.at[pl.ds(toff, WPB * RPW * ROW)], team_v)

        def team_sum(j):
            def body(r, acc):
                return acc + team_v[pl.ds(r * (RPW * ROW) + j * ROW, ROW)]
            return jnp.sum(lax.fori_loop(0, WPB, body, zero16))

        def radix_body(t, basev):
            shift = jnp.int32(30) - 2 * t
            cand1 = basev + (one << shift)
            cand2 = basev + (jnp.int32(2) << shift)
            cand3 = basev + (jnp.int32(3) << shift)

            def scan(g, accs):
                a1, a2, a3 = accs
                for u in range(U):
                    kv = key_v[pl.ds((g * U + u) * 16, 16)]
                    a1 = a1 + jnp.where(kv >= cand1, one, 0)
                    a2 = a2 + jnp.where(kv >= cand2, one, 0)
                    a3 = a3 + jnp.where(kv >= cand3, one, 0)
                return a1, a2, a3

            a1, a2, a3 = lax.fori_loop(0, NG, scan, (zero16, zero16, zero16))
            publish(t % 2, [a1, a2, a3])
            t1, t2, t3 = team_sum(0), team_sum(1), team_sum(2)
            return jnp.where(
                t3 >= k, cand3,
                jnp.where(t2 >= k, cand2, jnp.where(t1 >= k, cand1, basev)))

        v = lax.fori_loop(0, 16, radix_body, jnp.int32(_INT_MIN))

        def count_scan(g, accs):
            ag, at_ = accs
            for u in range(U):
                kv = key_v[pl.ds((g * U + u) * 16, 16)]
                ag = ag + jnp.where(kv > v, one, 0)
                at_ = at_ + jnp.where(kv == v, one, 0)
            return ag, at_

        accg, acct = lax.fori_loop(0, NG, count_scan, (zero16, zero16))
        publish(0, [accg, acct])
        r_need = jnp.int32(k) - team_sum(0)
        tie_local = jnp.sum(acct)

        def prefix_body(rr, acc):
            rowsum = jnp.sum(team_v[pl.ds(rr * (RPW * ROW) + ROW, ROW)])
            return acc + jnp.where(rr < slot, rowsum, jnp.int32(0))

        tie_before = lax.fori_loop(0, WPB, prefix_body, jnp.int32(0))
        q = jnp.minimum(jnp.maximum(r_need - tie_before, jnp.int32(0)),
                        tie_local)

        fone, fzero = jnp.float32(1.0), jnp.float32(0.0)

        def write_plain(_):
            def body(g, carry):
                for u in range(U):
                    i = g * U + u
                    kv = key_v[pl.ds(i * 16, 16)]
                    out_v[pl.ds(i * 16, 16)] = jnp.where(kv > v, fone, fzero)
                return carry
            return lax.fori_loop(0, NG, body, jnp.int32(0))

        def write_all_ties(_):
            def body(g, carry):
                for u in range(U):
                    i = g * U + u
                    kv = key_v[pl.ds(i * 16, 16)]
                    out_v[pl.ds(i * 16, 16)] = jnp.where(kv >= v, fone, fzero)
                return carry
            return lax.fori_loop(0, NG, body, jnp.int32(0))

        def write_cumsum(_):
            def body(i, run):
                kv = key_v[pl.ds(i * 16, 16)]
                tie = kv == v
                csum = lax.cumsum(jnp.where(tie, one, 0))
                accept = tie & ((run + csum) <= q)
                out_v[pl.ds(i * 16, 16)] = jnp.where(
                    (kv > v) | accept, fone, fzero)
                return run + jnp.max(csum)
            return lax.fori_loop(0, NV, body, jnp.int32(0))

        _ = lax.cond(
            q == 0, write_plain,
            lambda _: lax.cond(q == tie_local, write_all_ties,
                               write_cumsum, 0),
            0)

        pltpu.sync_copy(out_v, out_hbm.at[pl.ds(base, CH)])

    return sc_mask(imp_flat)


def _sc_multiply(spikes3, mask3, T, B, C, H, W):
    info = plsc.get_sparse_core_info()
    NC, NS = info.num_cores, info.num_subcores
    WPB = (NC * NS) // B
    RPW = (T * C) // WPB
    WCH = W // 16
    NVP = H * WCH
    mesh = plsc.VectorSubcoreMesh(core_axis_name="c", subcore_axis_name="s")

    @functools.partial(
        pl.kernel,
        mesh=mesh,
        compiler_params=pltpu.CompilerParams(needs_layout_passes=False),
        out_type=jax.ShapeDtypeStruct((T * B * C, H, W), jnp.float32),
        scratch_types=[
            pltpu.VMEM((NVP,), jnp.int32),
            pltpu.VMEM((H, W), jnp.float32),
            pltpu.VMEM((H, W), jnp.float32),
            pltpu.SemaphoreType.DMA,
            pltpu.SemaphoreType.DMA,
            pltpu.SemaphoreType.DMA,
            pltpu.SemaphoreType.DMA,
        ],
    )
    def sc_mul(spikes_hbm, mask_hbm, out_hbm, bits_v, d0, d1,
               isem0, isem1, osem0, osem1):
        c = lax.axis_index("c")
        s = lax.axis_index("s")
        batch = c * (B // NC) + s // WPB
        wslot = s % WPB
        lane = jax.lax.iota(jnp.int32, 16)
        one = jnp.int32(1)

        pltpu.sync_copy(mask_hbm.at[batch], d0)

        def compress_grp(gg, carry):
            wvec = jnp.zeros((16,), jnp.int32)
            for j in range(16):
                v = gg * 16 + j
                hh = v // WCH
                u = v % WCH
                mv = d0[hh, pl.ds(u * 16, 16)]
                w = jnp.sum(jnp.where(mv != jnp.float32(0.0),
                                      one << lane, jnp.int32(0)))
                wvec = jnp.where(lane == j,
                                 jnp.full((16,), w, jnp.int32), wvec)
            bits_v[pl.ds(gg * 16, 16)] = wvec
            return carry

        lax.fori_loop(0, NVP // 16, compress_grp, jnp.int32(0))

        def row_of(ch):
            jj = wslot * RPW + ch
            t = jj // C
            cch = jj % C
            return ((t * B) + batch) * C + cch

        bufs = (d0, d1)
        isems = (isem0, isem1)
        osems = (osem0, osem1)

        pltpu.async_copy(spikes_hbm.at[row_of(0)], d0, isem0)

        def step(g, carry):
            for b2 in range(2):
                ch = g * 2 + b2
                me, other = bufs[b2], bufs[1 - b2]

                @pl.when(ch + 1 < RPW)
                def _start_next():
                    @pl.when(ch >= 1)
                    def _drain_other_out():
                        pltpu.make_async_copy(
                            other, out_hbm.at[row_of(ch - 1)],
                            osems[1 - b2]).wait()
                    pltpu.async_copy(
                        spikes_hbm.at[row_of(ch + 1)], other, isems[1 - b2])

                pltpu.make_async_copy(
                    spikes_hbm.at[row_of(ch)], me, isems[b2]).wait()

                def mul_grp(gg, cc):
                    wvec = bits_v[pl.ds(gg * 16, 16)]
                    for j in range(16):
                        v = gg * 16 + j
                        hh = v // WCH
                        u = v % WCH
                        w = jnp.sum(jnp.where(lane == j, wvec,
                                              jnp.int32(0)))
                        w16 = jnp.full((16,), w, jnp.int32)
                        keep = ((w16 >> lane) & one) != jnp.int32(0)
                        me[hh, pl.ds(u * 16, 16)] = jnp.where(
                            keep, me[hh, pl.ds(u * 16, 16)],
                            jnp.float32(0.0))
                    return cc

                lax.fori_loop(0, NVP // 16, mul_grp, jnp.int32(0))
                pltpu.async_copy(me, out_hbm.at[row_of(ch)], osems[b2])
            return carry

        lax.fori_loop(0, RPW // 2, step, jnp.int32(0))
        pltpu.make_async_copy(
            d0, out_hbm.at[row_of(RPW - 2)], osem0).wait()
        pltpu.make_async_copy(
            d1, out_hbm.at[row_of(RPW - 1)], osem1).wait()

    return sc_mul(spikes3, mask3)


def kernel(spikes, importance, training):
    T, B, C, H, W = spikes.shape
    N = H * W
    k = max(1, int(_TARGET_RATE * N))
    mask = _build_mask_sc(importance.reshape(B * N), B, N, k)
    out = _sc_multiply(spikes.reshape(T * B * C, H, W),
                       mask.reshape(B, H, W), T, B, C, H, W)
    return out.reshape(T, B, C, H, W)


# speedup vs baseline: 2.7947x; 2.7947x over previous
import functools

import jax
import jax.numpy as jnp
from jax import lax
from jax.experimental import pallas as pl
from jax.experimental.pallas import tpu as pltpu
from jax.experimental.pallas import tpu_sc as plsc

_TARGET_RATE = 0.5
_INT_MIN = -2147483648


def _build_mask_sc(imp_flat, B, N, k):
    info = plsc.get_sparse_core_info()
    NC, NS = info.num_cores, info.num_subcores
    WPB = (NC * NS) // B
    CH = N // WPB
    NV = CH // 16
    U = 8 if NV % 8 == 0 else 1
    NG = NV // U
    ROW = 16
    RPW = 3
    mesh = plsc.VectorSubcoreMesh(core_axis_name="c", subcore_axis_name="s")

    @functools.partial(
        pl.kernel,
        mesh=mesh,
        compiler_params=pltpu.CompilerParams(needs_layout_passes=False),
        out_type=jax.ShapeDtypeStruct((B * N,), jnp.float32),
        scratch_types=[
            pltpu.VMEM((CH,), jnp.float32),
            pltpu.VMEM((CH,), jnp.int32),
            pltpu.VMEM((CH,), jnp.float32),
            pltpu.VMEM((RPW * ROW,), jnp.int32),
            pltpu.VMEM((WPB * RPW * ROW,), jnp.int32),
            pltpu.VMEM_SHARED((2 * NS * RPW * ROW,), jnp.int32),
        ],
    )
    def sc_mask(imp_hbm, out_hbm, x_v, key_v, out_v, stage_v, team_v, counts_sm):
        c = lax.axis_index("c")
        s = lax.axis_index("s")
        batch = c * (B // NC) + s // WPB
        slot = s % WPB
        team_lo = (s // WPB) * WPB
        base = batch * N + slot * CH

        pltpu.sync_copy(imp_hbm.at[pl.ds(base, CH)], x_v)

        def keys_body(g, carry):
            for u in range(U):
                i = g * U + u
                bits = lax.bitcast_convert_type(x_v[pl.ds(i * 16, 16)],
                                                jnp.int32)
                key_v[pl.ds(i * 16, 16)] = jnp.where(
                    bits >= 0, bits, jnp.int32(_INT_MIN) - bits)
            return carry

        lax.fori_loop(0, NG, keys_body, jnp.int32(0))

        one = jnp.int32(1)
        zero16 = jnp.zeros((16,), jnp.int32)

        def publish(parity, vecs):
            for j, vec in enumerate(vecs):
                stage_v[pl.ds(j * ROW, ROW)] = vec
            off = (parity * NS + s) * (RPW * ROW)
            pltpu.sync_copy(stage_v, counts_sm.at[pl.ds(off, RPW * ROW)])
            plsc.subcore_barrier()
            toff = (parity * NS + team_lo) * (RPW * ROW)
            pltpu.sync_copy(counts_sm.at[pl.ds(toff, WPB * RPW * ROW)], team_v)

        def team_sum(j):
            def body(r, acc):
                return acc + team_v[pl.ds(r * (RPW * ROW) + j * ROW, ROW)]
            return jnp.sum(lax.fori_loop(0, WPB, body, zero16))

        def radix_body(t, basev):
            shift = jnp.int32(30) - 2 * t
            cand1 = basev + (one << shift)
            cand2 = basev + (jnp.int32(2) << shift)
            cand3 = basev + (jnp.int32(3) << shift)

            def scan(g, accs):
                a1, a2, a3 = accs
                for u in range(U):
                    kv = key_v[pl.ds((g * U + u) * 16, 16)]
                    a1 = a1 + jnp.where(kv >= cand1, one, 0)
                    a2 = a2 + jnp.where(kv >= cand2, one, 0)
                    a3 = a3 + jnp.where(kv >= cand3, one, 0)
                return a1, a2, a3

            a1, a2, a3 = lax.fori_loop(0, NG, scan, (zero16, zero16, zero16))
            publish(t % 2, [a1, a2, a3])
            t1, t2, t3 = team_sum(0), team_sum(1), team_sum(2)
            return jnp.where(
                t3 >= k, cand3,
                jnp.where(t2 >= k, cand2, jnp.where(t1 >= k, cand1, basev)))

        v = lax.fori_loop(0, 16, radix_body, jnp.int32(_INT_MIN))

        def count_scan(g, accs):
            ag, at_ = accs
            for u in range(U):
                kv = key_v[pl.ds((g * U + u) * 16, 16)]
                ag = ag + jnp.where(kv > v, one, 0)
                at_ = at_ + jnp.where(kv == v, one, 0)
            return ag, at_

        accg, acct = lax.fori_loop(0, NG, count_scan, (zero16, zero16))
        publish(0, [accg, acct])
        r_need = jnp.int32(k) - team_sum(0)
        tie_local = jnp.sum(acct)

        def prefix_body(rr, acc):
            rowsum = jnp.sum(team_v[pl.ds(rr * (RPW * ROW) + ROW, ROW)])
            return acc + jnp.where(rr < slot, rowsum, jnp.int32(0))

        tie_before = lax.fori_loop(0, WPB, prefix_body, jnp.int32(0))
        q = jnp.minimum(jnp.maximum(r_need - tie_before, jnp.int32(0)),
                        tie_local)

        fone, fzero = jnp.float32(1.0), jnp.float32(0.0)

        def write_plain(_):
            def body(g, carry):
                for u in range(U):
                    i = g * U + u
                    kv = key_v[pl.ds(i * 16, 16)]
                    out_v[pl.ds(i * 16, 16)] = jnp.where(kv > v, fone, fzero)
                return carry
            return lax.fori_loop(0, NG, body, jnp.int32(0))

        def write_all_ties(_):
            def body(g, carry):
                for u in range(U):
                    i = g * U + u
                    kv = key_v[pl.ds(i * 16, 16)]
                    out_v[pl.ds(i * 16, 16)] = jnp.where(kv >= v, fone, fzero)
                return carry
            return lax.fori_loop(0, NG, body, jnp.int32(0))

        def write_cumsum(_):
            def body(i, run):
                kv = key_v[pl.ds(i * 16, 16)]
                tie = kv == v
                csum = lax.cumsum(jnp.where(tie, one, 0))
                accept = tie & ((run + csum) <= q)
                out_v[pl.ds(i * 16, 16)] = jnp.where(
                    (kv > v) | accept, fone, fzero)
                return run + jnp.max(csum)
            return lax.fori_loop(0, NV, body, jnp.int32(0))

        _ = lax.cond(
            q == 0, write_plain,
            lambda _: lax.cond(q == tie_local, write_all_ties,
                               write_cumsum, 0),
            0)

        pltpu.sync_copy(out_v, out_hbm.at[pl.ds(base, CH)])

    return sc_mask(imp_flat)


def _sc_multiply(spikes3, mask3, T, B, C, H, W):
    info = plsc.get_sparse_core_info()
    NC, NS = info.num_cores, info.num_subcores
    WPB = (NC * NS) // B
    RPW = (T * C) // WPB
    WCH = W // 16
    NVP = H * WCH
    mesh = plsc.VectorSubcoreMesh(core_axis_name="c", subcore_axis_name="s")

    @functools.partial(
        pl.kernel,
        mesh=mesh,
        compiler_params=pltpu.CompilerParams(needs_layout_passes=False),
        out_type=jax.ShapeDtypeStruct((T * B * C, H, W), jnp.float32),
        scratch_types=[
            pltpu.VMEM(((H // 32) * WCH * 16,), jnp.int32),
            pltpu.VMEM((H, W), jnp.float32),
            pltpu.VMEM((H, W), jnp.float32),
            pltpu.SemaphoreType.DMA,
            pltpu.SemaphoreType.DMA,
            pltpu.SemaphoreType.DMA,
            pltpu.SemaphoreType.DMA,
        ],
    )
    def sc_mul(spikes_hbm, mask_hbm, out_hbm, bits_v, d0, d1,
               isem0, isem1, osem0, osem1):
        c = lax.axis_index("c")
        s = lax.axis_index("s")
        batch = c * (B // NC) + s // WPB
        wslot = s % WPB
        lane = jax.lax.iota(jnp.int32, 16)
        one = jnp.int32(1)

        pltpu.sync_copy(mask_hbm.at[batch], d0)

        def cmp_body(idx, carry):
            rb = idx // WCH
            u = idx % WCH
            acc = jnp.zeros((16,), jnp.int32)
            for r in range(32):
                cbit = (1 << r) & 0xFFFFFFFF
                if cbit >= 2**31:
                    cbit -= 2**32
                mv = d0[rb * 32 + r, pl.ds(u * 16, 16)]
                acc = acc | jnp.where(mv != jnp.float32(0.0),
                                      jnp.int32(cbit), jnp.int32(0))
            bits_v[pl.ds(idx * 16, 16)] = acc
            return carry

        lax.fori_loop(0, (H // 32) * WCH, cmp_body, jnp.int32(0))

        def row_of(ch):
            jj = wslot * RPW + ch
            t = jj // C
            cch = jj % C
            return ((t * B) + batch) * C + cch

        bufs = (d0, d1)
        isems = (isem0, isem1)
        osems = (osem0, osem1)

        pltpu.async_copy(spikes_hbm.at[row_of(0)], d0, isem0)

        def step(g, carry):
            for b2 in range(2):
                ch = g * 2 + b2
                me, other = bufs[b2], bufs[1 - b2]

                @pl.when(ch + 1 < RPW)
                def _start_next():
                    @pl.when(ch >= 1)
                    def _drain_other_out():
                        pltpu.make_async_copy(
                            other, out_hbm.at[row_of(ch - 1)],
                            osems[1 - b2]).wait()
                    pltpu.async_copy(
                        spikes_hbm.at[row_of(ch + 1)], other, isems[1 - b2])

                pltpu.make_async_copy(
                    spikes_hbm.at[row_of(ch)], me, isems[b2]).wait()

                def mul_blk(idx, cc):
                    rb = idx // WCH
                    u = idx % WCH
                    bv = bits_v[pl.ds(idx * 16, 16)]
                    for r in range(32):
                        hh = rb * 32 + r
                        keep = ((bv >> r) & one) != jnp.int32(0)
                        me[hh, pl.ds(u * 16, 16)] = jnp.where(
                            keep, me[hh, pl.ds(u * 16, 16)],
                            jnp.float32(0.0))
                    return cc

                lax.fori_loop(0, (H // 32) * WCH, mul_blk, jnp.int32(0))
                pltpu.async_copy(me, out_hbm.at[row_of(ch)], osems[b2])
            return carry

        lax.fori_loop(0, RPW // 2, step, jnp.int32(0))
        pltpu.make_async_copy(
            d0, out_hbm.at[row_of(RPW - 2)], osem0).wait()
        pltpu.make_async_copy(
            d1, out_hbm.at[row_of(RPW - 1)], osem1).wait()

    return sc_mul(spikes3, mask3)


def kernel(spikes, importance, training):
    T, B, C, H, W = spikes.shape
    N = H * W
    k = max(1, int(_TARGET_RATE * N))
    mask = _build_mask_sc(importance.reshape(B * N), B, N, k)
    out = _sc_multiply(spikes.reshape(T * B * C, H, W),
                       mask.reshape(B, H, W), T, B, C, H, W)
    return out.reshape(T, B, C, H, W)
